# baseline (speedup 1.0000x reference)
import functools

import jax
import jax.numpy as jnp
from jax import lax
from jax.experimental import pallas as pl
from jax.experimental.pallas import tpu as pltpu
from jax.experimental.pallas import tpu_sc as plsc

_CHUNK = 128
_NBUF = 3
_AHEAD = 2


_REP = 8


def _table_body(bt_ref, pi_ref, out_ref):
    bt = bt_ref[...]
    e = jnp.exp(bt - jnp.max(bt, axis=0, keepdims=True))
    s = jnp.sum(e, axis=0, keepdims=True)
    pi = pi_ref[...]
    pe = jnp.exp(pi - jnp.max(pi, axis=1, keepdims=True))
    ps = jnp.sum(pe, axis=1, keepdims=True)
    w = e * (pe / (s * ps))
    out_ref[...] = jnp.concatenate([w] * _REP, axis=1)


def kernel(x, B, Pi):
    c_dim, m_dim = B.shape
    n_dim = x.shape[0]
    ch = _CHUNK
    n_chunks = -(-n_dim // ch)

    wt = pl.pallas_call(
        _table_body,
        out_shape=jax.ShapeDtypeStruct((m_dim, _REP * c_dim), jnp.float32),
    )(B.T, Pi.reshape(1, c_dim))
    wt = wt.reshape(m_dim * _REP, c_dim)

    mesh = plsc.VectorSubcoreMesh(core_axis_name="c", subcore_axis_name="s")
    nw = mesh.num_cores * mesh.num_subcores
    n_iters = -(-n_chunks // nw)
    nbuf = _NBUF
    ahead = _AHEAD
    n_outer = -(-n_iters // nbuf)
    assert n_iters >= nbuf

    @functools.partial(
        pl.kernel,
        out_type=jax.ShapeDtypeStruct((n_dim, c_dim), jnp.float32),
        mesh=mesh,
        scratch_types=[
            [pltpu.VMEM((ch,), jnp.int32)] * nbuf,
            [pltpu.VMEM((ch, c_dim), jnp.float32)] * nbuf,
            [pltpu.SemaphoreType.DMA] * nbuf,
            [pltpu.SemaphoreType.DMA] * nbuf,
            [pltpu.SemaphoreType.DMA] * nbuf,
        ],
    )
    def _gather(wt_hbm, x_hbm, out_hbm, idx, rows, isem, gsem, osem):
        cid = lax.axis_index("c")
        sid = lax.axis_index("s")
        wid = sid * mesh.num_cores + cid

        def adjust(b):
            rep = lax.iota(jnp.int32, 16) & (_REP - 1)
            for i in range(ch // 16):
                v = idx[b][pl.ds(16 * i, 16)]
                idx[b][pl.ds(16 * i, 16)] = v * _REP + rep

        def active(c):
            return (wid + c * nw) < n_chunks

        def off_of(c):
            return jnp.minimum((wid + c * nw) * ch, n_dim - ch)

        for b in range(nbuf):
            pltpu.async_copy(x_hbm.at[pl.ds(off_of(b), ch)], idx[b], isem[b])
        for b in range(ahead):
            pltpu.make_async_copy(
                x_hbm.at[pl.ds(off_of(b), ch)], idx[b], isem[b]).wait()
            adjust(b)
            pltpu.async_copy(wt_hbm.at[idx[b]], rows[b], gsem[b])

        def body(jq, carry):
            for b in range(nbuf):
                j = nbuf * jq + b

                @pl.when(active(j))
                def _(b=b, j=j):
                    off = off_of(j)
                    pltpu.make_async_copy(
                        wt_hbm.at[idx[b]], rows[b], gsem[b]).wait()
                    pltpu.async_copy(
                        rows[b], out_hbm.at[pl.ds(off, ch)], osem[b])

                    @pl.when(active(j + nbuf))
                    def _():
                        pltpu.async_copy(
                            x_hbm.at[pl.ds(off_of(j + nbuf), ch)],
                            idx[b], isem[b])

                b3 = (b + ahead) % nbuf
                c3 = j + ahead

                @pl.when(active(c3))
                def _(b3=b3, c3=c3, b=b, jq=jq):
                    pltpu.make_async_copy(
                        x_hbm.at[pl.ds(off_of(c3), ch)], idx[b3], isem[b3]
                    ).wait()
                    adjust(b3)

                    def wait_prev_store():
                        pltpu.make_async_copy(
                            rows[b3], out_hbm.at[pl.ds(0, ch)], osem[b3]
                        ).wait()

                    if b + ahead >= nbuf:
                        wait_prev_store()
                    else:
                        pl.when(jq >= 1)(wait_prev_store)
                    pltpu.async_copy(wt_hbm.at[idx[b3]], rows[b3], gsem[b3])

            return carry

        lax.fori_loop(0, n_outer, body, 0)

        for b in range(nbuf):
            pltpu.make_async_copy(
                rows[b], out_hbm.at[pl.ds(0, ch)], osem[b]).wait()

    return _gather(wt, x)


# speedup vs baseline: 1.0301x; 1.0301x over previous
import functools

import jax
import jax.numpy as jnp
from jax import lax
from jax.experimental import pallas as pl
from jax.experimental.pallas import tpu as pltpu
from jax.experimental.pallas import tpu_sc as plsc

_CHUNK = 128
_NBUF = 3
_AHEAD = 2


_REP = 8


def _table_body(bt_ref, pi_ref, out_ref):
    bt = bt_ref[...]
    e = jnp.exp(bt - jnp.max(bt, axis=0, keepdims=True))
    s = jnp.sum(e, axis=0, keepdims=True)
    pi = pi_ref[...]
    pe = jnp.exp(pi - jnp.max(pi, axis=1, keepdims=True))
    ps = jnp.sum(pe, axis=1, keepdims=True)
    out_ref[...] = e * (pe / (s * ps))


def kernel(x, B, Pi):
    c_dim, m_dim = B.shape
    n_dim = x.shape[0]
    ch = _CHUNK
    n_chunks = -(-n_dim // ch)

    wt = pl.pallas_call(
        _table_body,
        grid=(_REP,),
        in_specs=[
            pl.BlockSpec((m_dim, c_dim), lambda r: (0, 0)),
            pl.BlockSpec((1, c_dim), lambda r: (0, 0)),
        ],
        out_specs=pl.BlockSpec((m_dim, c_dim), lambda r: (r, 0)),
        out_shape=jax.ShapeDtypeStruct((_REP * m_dim, c_dim), jnp.float32),
    )(B.T, Pi.reshape(1, c_dim))

    mesh = plsc.VectorSubcoreMesh(core_axis_name="c", subcore_axis_name="s")
    nw = mesh.num_cores * mesh.num_subcores
    n_iters = -(-n_chunks // nw)
    nbuf = _NBUF
    ahead = _AHEAD
    n_outer = -(-n_iters // nbuf)
    assert n_iters >= nbuf

    @functools.partial(
        pl.kernel,
        out_type=jax.ShapeDtypeStruct((n_dim, c_dim), jnp.float32),
        mesh=mesh,
        scratch_types=[
            [pltpu.VMEM((ch,), jnp.int32)] * nbuf,
            [pltpu.VMEM((ch, c_dim), jnp.float32)] * nbuf,
            [pltpu.SemaphoreType.DMA] * nbuf,
            [pltpu.SemaphoreType.DMA] * nbuf,
            [pltpu.SemaphoreType.DMA] * nbuf,
        ],
    )
    def _gather(wt_hbm, x_hbm, out_hbm, idx, rows, isem, gsem, osem):
        cid = lax.axis_index("c")
        sid = lax.axis_index("s")
        wid = sid * mesh.num_cores + cid

        def adjust(b):
            rep = (lax.iota(jnp.int32, 16) & (_REP - 1)) * m_dim
            for i in range(ch // 16):
                v = idx[b][pl.ds(16 * i, 16)]
                idx[b][pl.ds(16 * i, 16)] = v + rep

        def active(c):
            return (wid + c * nw) < n_chunks

        def off_of(c):
            return jnp.minimum((wid + c * nw) * ch, n_dim - ch)

        for b in range(nbuf):
            pltpu.async_copy(x_hbm.at[pl.ds(off_of(b), ch)], idx[b], isem[b])
        for b in range(ahead):
            pltpu.make_async_copy(
                x_hbm.at[pl.ds(off_of(b), ch)], idx[b], isem[b]).wait()
            adjust(b)
            pltpu.async_copy(wt_hbm.at[idx[b]], rows[b], gsem[b])

        def body(jq, carry):
            for b in range(nbuf):
                j = nbuf * jq + b

                @pl.when(active(j))
                def _(b=b, j=j):
                    off = off_of(j)
                    pltpu.make_async_copy(
                        wt_hbm.at[idx[b]], rows[b], gsem[b]).wait()
                    pltpu.async_copy(
                        rows[b], out_hbm.at[pl.ds(off, ch)], osem[b])

                    @pl.when(active(j + nbuf))
                    def _():
                        pltpu.async_copy(
                            x_hbm.at[pl.ds(off_of(j + nbuf), ch)],
                            idx[b], isem[b])

                b3 = (b + ahead) % nbuf
                c3 = j + ahead

                @pl.when(active(c3))
                def _(b3=b3, c3=c3, b=b, jq=jq):
                    pltpu.make_async_copy(
                        x_hbm.at[pl.ds(off_of(c3), ch)], idx[b3], isem[b3]
                    ).wait()
                    adjust(b3)

                    def wait_prev_store():
                        pltpu.make_async_copy(
                            rows[b3], out_hbm.at[pl.ds(0, ch)], osem[b3]
                        ).wait()

                    if b + ahead >= nbuf:
                        wait_prev_store()
                    else:
                        pl.when(jq >= 1)(wait_prev_store)
                    pltpu.async_copy(wt_hbm.at[idx[b3]], rows[b3], gsem[b3])

            return carry

        lax.fori_loop(0, n_outer, body, 0)

        for b in range(nbuf):
            pltpu.make_async_copy(
                rows[b], out_hbm.at[pl.ds(0, ch)], osem[b]).wait()

    return _gather(wt, x)


# speedup vs baseline: 1.0567x; 1.0258x over previous
import functools

import jax
import jax.numpy as jnp
from jax import lax
from jax.experimental import pallas as pl
from jax.experimental.pallas import tpu as pltpu
from jax.experimental.pallas import tpu_sc as plsc

_CHUNK = 128
_NBUF = 3
_AHEAD = 2


_REP = 8


def _table_body(b_ref, pi_ref, out_ref):
    b = b_ref[...]
    e = jnp.exp(b - jnp.max(b, axis=1, keepdims=True))
    s = jnp.sum(e, axis=1, keepdims=True)
    pi = pi_ref[...]
    pe = jnp.exp(pi - jnp.max(pi, axis=0, keepdims=True))
    ps = jnp.sum(pe, axis=0, keepdims=True)
    w = (e * (pe / (s * ps))).T
    m_dim = w.shape[0]
    for r in range(_REP):
        out_ref[pl.ds(r * m_dim, m_dim), :] = w


def kernel(x, B, Pi):
    c_dim, m_dim = B.shape
    n_dim = x.shape[0]
    ch = _CHUNK
    n_chunks = -(-n_dim // ch)

    wt = pl.pallas_call(
        _table_body,
        out_shape=jax.ShapeDtypeStruct((_REP * m_dim, c_dim), jnp.float32),
    )(B, Pi.reshape(c_dim, 1))

    mesh = plsc.VectorSubcoreMesh(core_axis_name="c", subcore_axis_name="s")
    nw = mesh.num_cores * mesh.num_subcores
    n_iters = -(-n_chunks // nw)
    nbuf = _NBUF
    ahead = _AHEAD
    n_outer = -(-n_iters // nbuf)
    assert n_iters >= nbuf

    @functools.partial(
        pl.kernel,
        out_type=jax.ShapeDtypeStruct((n_dim, c_dim), jnp.float32),
        mesh=mesh,
        scratch_types=[
            [pltpu.VMEM((ch,), jnp.int32)] * nbuf,
            [pltpu.VMEM((ch, c_dim), jnp.float32)] * nbuf,
            [pltpu.SemaphoreType.DMA] * nbuf,
            [pltpu.SemaphoreType.DMA] * nbuf,
            [pltpu.SemaphoreType.DMA] * nbuf,
        ],
    )
    def _gather(wt_hbm, x_hbm, out_hbm, idx, rows, isem, gsem, osem):
        cid = lax.axis_index("c")
        sid = lax.axis_index("s")
        wid = sid * mesh.num_cores + cid

        def adjust(b):
            rep = (lax.iota(jnp.int32, 16) & (_REP - 1)) * m_dim
            for i in range(ch // 16):
                v = idx[b][pl.ds(16 * i, 16)]
                idx[b][pl.ds(16 * i, 16)] = v + rep

        def active(c):
            return (wid + c * nw) < n_chunks

        def off_of(c):
            return jnp.minimum((wid + c * nw) * ch, n_dim - ch)

        for b in range(nbuf):
            pltpu.async_copy(x_hbm.at[pl.ds(off_of(b), ch)], idx[b], isem[b])
        for b in range(ahead):
            pltpu.make_async_copy(
                x_hbm.at[pl.ds(off_of(b), ch)], idx[b], isem[b]).wait()
            adjust(b)
            pltpu.async_copy(wt_hbm.at[idx[b]], rows[b], gsem[b])

        def body(jq, carry):
            for b in range(nbuf):
                j = nbuf * jq + b

                @pl.when(active(j))
                def _(b=b, j=j):
                    off = off_of(j)
                    pltpu.make_async_copy(
                        wt_hbm.at[idx[b]], rows[b], gsem[b]).wait()
                    pltpu.async_copy(
                        rows[b], out_hbm.at[pl.ds(off, ch)], osem[b])

                    @pl.when(active(j + nbuf))
                    def _():
                        pltpu.async_copy(
                            x_hbm.at[pl.ds(off_of(j + nbuf), ch)],
                            idx[b], isem[b])

                b3 = (b + ahead) % nbuf
                c3 = j + ahead

                @pl.when(active(c3))
                def _(b3=b3, c3=c3, b=b, jq=jq):
                    pltpu.make_async_copy(
                        x_hbm.at[pl.ds(off_of(c3), ch)], idx[b3], isem[b3]
                    ).wait()
                    adjust(b3)

                    def wait_prev_store():
                        pltpu.make_async_copy(
                            rows[b3], out_hbm.at[pl.ds(0, ch)], osem[b3]
                        ).wait()

                    if b + ahead >= nbuf:
                        wait_prev_store()
                    else:
                        pl.when(jq >= 1)(wait_prev_store)
                    pltpu.async_copy(wt_hbm.at[idx[b3]], rows[b3], gsem[b3])

            return carry

        lax.fori_loop(0, n_outer, body, 0)

        for b in range(nbuf):
            pltpu.make_async_copy(
                rows[b], out_hbm.at[pl.ds(0, ch)], osem[b]).wait()

    return _gather(wt, x)


# speedup vs baseline: 1.0729x; 1.0154x over previous
import functools

import jax
import jax.numpy as jnp
from jax import lax
from jax.experimental import pallas as pl
from jax.experimental.pallas import tpu as pltpu
from jax.experimental.pallas import tpu_sc as plsc

_CHUNK = 128
_NBUF = 3
_AHEAD = 2


_REP = 16


def _table_body(b_ref, pi_ref, out_ref):
    b = b_ref[...]
    e = jnp.exp(b - jnp.max(b, axis=1, keepdims=True))
    s = jnp.sum(e, axis=1, keepdims=True)
    pi = pi_ref[...]
    pe = jnp.exp(pi - jnp.max(pi, axis=0, keepdims=True))
    ps = jnp.sum(pe, axis=0, keepdims=True)
    w = (e * (pe / (s * ps))).T
    m_dim = w.shape[0]
    for r in range(_REP):
        out_ref[pl.ds(r * m_dim, m_dim), :] = w


def kernel(x, B, Pi):
    c_dim, m_dim = B.shape
    n_dim = x.shape[0]
    ch = _CHUNK
    n_chunks = -(-n_dim // ch)

    wt = pl.pallas_call(
        _table_body,
        out_shape=jax.ShapeDtypeStruct((_REP * m_dim, c_dim), jnp.float32),
    )(B, Pi.reshape(c_dim, 1))

    mesh = plsc.VectorSubcoreMesh(core_axis_name="c", subcore_axis_name="s")
    nw = mesh.num_cores * mesh.num_subcores
    n_iters = -(-n_chunks // nw)
    nbuf = _NBUF
    ahead = _AHEAD
    n_outer = -(-n_iters // nbuf)
    assert n_iters >= nbuf

    @functools.partial(
        pl.kernel,
        out_type=jax.ShapeDtypeStruct((n_dim, c_dim), jnp.float32),
        mesh=mesh,
        scratch_types=[
            [pltpu.VMEM((ch,), jnp.int32)] * nbuf,
            [pltpu.VMEM((ch, c_dim), jnp.float32)] * nbuf,
            [pltpu.SemaphoreType.DMA] * nbuf,
            [pltpu.SemaphoreType.DMA] * nbuf,
            [pltpu.SemaphoreType.DMA] * nbuf,
        ],
    )
    def _gather(wt_hbm, x_hbm, out_hbm, idx, rows, isem, gsem, osem):
        cid = lax.axis_index("c")
        sid = lax.axis_index("s")
        wid = sid * mesh.num_cores + cid

        def adjust(b):
            rep = (lax.iota(jnp.int32, 16) & (_REP - 1)) * m_dim
            for i in range(ch // 16):
                v = idx[b][pl.ds(16 * i, 16)]
                idx[b][pl.ds(16 * i, 16)] = v + rep

        def active(c):
            return (wid + c * nw) < n_chunks

        def off_of(c):
            return jnp.minimum((wid + c * nw) * ch, n_dim - ch)

        for b in range(nbuf):
            pltpu.async_copy(x_hbm.at[pl.ds(off_of(b), ch)], idx[b], isem[b])
        for b in range(ahead):
            pltpu.make_async_copy(
                x_hbm.at[pl.ds(off_of(b), ch)], idx[b], isem[b]).wait()
            adjust(b)
            pltpu.async_copy(wt_hbm.at[idx[b]], rows[b], gsem[b])

        def body(jq, carry):
            for b in range(nbuf):
                j = nbuf * jq + b

                @pl.when(active(j))
                def _(b=b, j=j):
                    off = off_of(j)
                    pltpu.make_async_copy(
                        wt_hbm.at[idx[b]], rows[b], gsem[b]).wait()
                    pltpu.async_copy(
                        rows[b], out_hbm.at[pl.ds(off, ch)], osem[b])

                    @pl.when(active(j + nbuf))
                    def _():
                        pltpu.async_copy(
                            x_hbm.at[pl.ds(off_of(j + nbuf), ch)],
                            idx[b], isem[b])

                b3 = (b + ahead) % nbuf
                c3 = j + ahead

                @pl.when(active(c3))
                def _(b3=b3, c3=c3, b=b, jq=jq):
                    pltpu.make_async_copy(
                        x_hbm.at[pl.ds(off_of(c3), ch)], idx[b3], isem[b3]
                    ).wait()
                    adjust(b3)

                    def wait_prev_store():
                        pltpu.make_async_copy(
                            rows[b3], out_hbm.at[pl.ds(0, ch)], osem[b3]
                        ).wait()

                    if b + ahead >= nbuf:
                        wait_prev_store()
                    else:
                        pl.when(jq >= 1)(wait_prev_store)
                    pltpu.async_copy(wt_hbm.at[idx[b3]], rows[b3], gsem[b3])

            return carry

        lax.fori_loop(0, n_outer, body, 0)

        for b in range(nbuf):
            pltpu.make_async_copy(
                rows[b], out_hbm.at[pl.ds(0, ch)], osem[b]).wait()

    return _gather(wt, x)
